# Initial kernel scaffold; baseline (speedup 1.0000x reference)
import jax
import jax.numpy as jnp
from jax.experimental import pallas as pl


def kernel(rgb_ids, ir_ids, class_confidence_v, class_confidence_r):
    raise NotImplementedError("write your pallas kernel here")


# speedup vs baseline: 1.3483x; 1.3483x over previous
import functools

import jax
import jax.numpy as jnp
from jax import lax
from jax.experimental import pallas as pl
from jax.experimental.pallas import tpu as pltpu
from jax.experimental.pallas import tpu_sc as plsc

_BATCH = 16384
_ROWS = 128
_COLS = 128
_NUM_WORKERS = 32
_ROWS_PER_WORKER = _ROWS // _NUM_WORKERS


def _gather_body(rgb_hbm, ir_hbm, table_v_hbm, table_r_hbm,
                 out_v_hbm, out_r_hbm,
                 idx_v, idx_r, rows_v, rows_r, sem_v, sem_r):
    wid = lax.axis_index("s") * 2 + lax.axis_index("c")
    base = wid * _ROWS_PER_WORKER
    pltpu.sync_copy(rgb_hbm.at[pl.ds(base, _ROWS_PER_WORKER)], idx_v)
    pltpu.sync_copy(ir_hbm.at[pl.ds(base, _ROWS_PER_WORKER)], idx_r)
    copies = []
    for j in range(_ROWS_PER_WORKER):
        copies.append(
            pltpu.async_copy(table_v_hbm.at[idx_v.at[j]], rows_v.at[j], sem_v))
        copies.append(
            pltpu.async_copy(table_r_hbm.at[idx_r.at[j]], rows_r.at[j], sem_r))
    for cp in copies:
        cp.wait()
    pltpu.sync_copy(rows_v, out_v_hbm.at[pl.ds(base, _ROWS_PER_WORKER)])
    pltpu.sync_copy(rows_r, out_r_hbm.at[pl.ds(base, _ROWS_PER_WORKER)])


@jax.jit
def kernel(rgb_ids, ir_ids, class_confidence_v, class_confidence_r):
    rgb2 = rgb_ids.astype(jnp.int32).reshape(_ROWS, _COLS)
    ir2 = ir_ids.astype(jnp.int32).reshape(_ROWS, _COLS)
    mesh = plsc.VectorSubcoreMesh(core_axis_name="c", subcore_axis_name="s")
    f = functools.partial(
        pl.kernel,
        mesh=mesh,
        out_type=(
            jax.ShapeDtypeStruct((_ROWS, _COLS), jnp.float32),
            jax.ShapeDtypeStruct((_ROWS, _COLS), jnp.float32),
        ),
        scratch_types=[
            pltpu.VMEM((_ROWS_PER_WORKER, _COLS), jnp.int32),
            pltpu.VMEM((_ROWS_PER_WORKER, _COLS), jnp.int32),
            pltpu.VMEM((_ROWS_PER_WORKER, _COLS), jnp.float32),
            pltpu.VMEM((_ROWS_PER_WORKER, _COLS), jnp.float32),
            pltpu.SemaphoreType.DMA,
            pltpu.SemaphoreType.DMA,
        ],
    )(_gather_body)
    out_v, out_r = f(rgb2, ir2, class_confidence_v, class_confidence_r)
    return out_v.reshape(_BATCH), out_r.reshape(_BATCH)


# speedup vs baseline: 1.3547x; 1.0048x over previous
import functools

import jax
import jax.numpy as jnp
from jax import lax
from jax.experimental import pallas as pl
from jax.experimental.pallas import tpu as pltpu
from jax.experimental.pallas import tpu_sc as plsc

_BATCH = 16384
_NUM_WORKERS = 32
_IDS_PER_WORKER = _BATCH // _NUM_WORKERS
_IDS_PER_STREAM = 128
_STREAMS = _IDS_PER_WORKER // _IDS_PER_STREAM


def _gather_body(rgb_hbm, ir_hbm, table_v_hbm, table_r_hbm,
                 out_v_hbm, out_r_hbm,
                 idx_v, idx_r, rows_v, rows_r, sem_i, sem_g):
    wid = lax.axis_index("s") * 2 + lax.axis_index("c")
    base = wid * _IDS_PER_WORKER
    cp_iv = pltpu.async_copy(rgb_hbm.at[pl.ds(base, _IDS_PER_WORKER)], idx_v,
                             sem_i)
    cp_ir = pltpu.async_copy(ir_hbm.at[pl.ds(base, _IDS_PER_WORKER)], idx_r,
                             sem_i)
    cp_iv.wait()
    cp_ir.wait()
    copies = []
    for j in range(_STREAMS):
        sl = pl.ds(j * _IDS_PER_STREAM, _IDS_PER_STREAM)
        copies.append(
            pltpu.async_copy(table_v_hbm.at[idx_v.at[sl]], rows_v.at[sl],
                             sem_g))
        copies.append(
            pltpu.async_copy(table_r_hbm.at[idx_r.at[sl]], rows_r.at[sl],
                             sem_g))
    for cp in copies:
        cp.wait()
    pltpu.sync_copy(rows_v, out_v_hbm.at[pl.ds(base, _IDS_PER_WORKER)])
    pltpu.sync_copy(rows_r, out_r_hbm.at[pl.ds(base, _IDS_PER_WORKER)])


@jax.jit
def kernel(rgb_ids, ir_ids, class_confidence_v, class_confidence_r):
    mesh = plsc.VectorSubcoreMesh(core_axis_name="c", subcore_axis_name="s")
    f = functools.partial(
        pl.kernel,
        mesh=mesh,
        out_type=(
            jax.ShapeDtypeStruct((_BATCH,), jnp.float32),
            jax.ShapeDtypeStruct((_BATCH,), jnp.float32),
        ),
        scratch_types=[
            pltpu.VMEM((_IDS_PER_WORKER,), jnp.int32),
            pltpu.VMEM((_IDS_PER_WORKER,), jnp.int32),
            pltpu.VMEM((_IDS_PER_WORKER,), jnp.float32),
            pltpu.VMEM((_IDS_PER_WORKER,), jnp.float32),
            pltpu.SemaphoreType.DMA,
            pltpu.SemaphoreType.DMA,
        ],
    )(_gather_body)
    return f(rgb_ids.astype(jnp.int32), ir_ids.astype(jnp.int32),
             class_confidence_v, class_confidence_r)


# speedup vs baseline: 1.3581x; 1.0025x over previous
import functools

import jax
import jax.numpy as jnp
from jax import lax
from jax.experimental import pallas as pl
from jax.experimental.pallas import tpu as pltpu
from jax.experimental.pallas import tpu_sc as plsc

_BATCH = 16384
_NUM_WORKERS = 32
_IDS_PER_WORKER = _BATCH // _NUM_WORKERS


def _gather_body(rgb_hbm, ir_hbm, table_v_hbm, table_r_hbm,
                 out_v_hbm, out_r_hbm,
                 idx_v, idx_r, rows_v, rows_r, sem_i, sem_g, sem_o):
    wid = lax.axis_index("s") * 2 + lax.axis_index("c")
    sl = pl.ds(wid * _IDS_PER_WORKER, _IDS_PER_WORKER)
    cp_iv = pltpu.async_copy(rgb_hbm.at[sl], idx_v, sem_i)
    cp_ir = pltpu.async_copy(ir_hbm.at[sl], idx_r, sem_i)
    cp_iv.wait()
    cp_gv = pltpu.async_copy(table_v_hbm.at[idx_v], rows_v, sem_g)
    cp_ir.wait()
    cp_gr = pltpu.async_copy(table_r_hbm.at[idx_r], rows_r, sem_g)
    cp_gv.wait()
    cp_ov = pltpu.async_copy(rows_v, out_v_hbm.at[sl], sem_o)
    cp_gr.wait()
    cp_or = pltpu.async_copy(rows_r, out_r_hbm.at[sl], sem_o)
    cp_ov.wait()
    cp_or.wait()


@jax.jit
def kernel(rgb_ids, ir_ids, class_confidence_v, class_confidence_r):
    mesh = plsc.VectorSubcoreMesh(core_axis_name="c", subcore_axis_name="s")
    f = functools.partial(
        pl.kernel,
        mesh=mesh,
        out_type=(
            jax.ShapeDtypeStruct((_BATCH,), jnp.float32),
            jax.ShapeDtypeStruct((_BATCH,), jnp.float32),
        ),
        scratch_types=[
            pltpu.VMEM((_IDS_PER_WORKER,), jnp.int32),
            pltpu.VMEM((_IDS_PER_WORKER,), jnp.int32),
            pltpu.VMEM((_IDS_PER_WORKER,), jnp.float32),
            pltpu.VMEM((_IDS_PER_WORKER,), jnp.float32),
            pltpu.SemaphoreType.DMA,
            pltpu.SemaphoreType.DMA,
            pltpu.SemaphoreType.DMA,
        ],
    )(_gather_body)
    return f(rgb_ids.astype(jnp.int32), ir_ids.astype(jnp.int32),
             class_confidence_v, class_confidence_r)
